# baseline (speedup 1.0000x reference)
import jax
import jax.numpy as jnp
from jax.experimental import pallas as pl
from jax.experimental.pallas import tpu as pltpu

_E = 5
_D = 32
_ED = _E * _D
_EPS = 1e-5
_BRANCH_WEIGHT = 0.92


def _branch_block(sp_ref, x_ref, w1_ref, b1_ref, w2_ref, b2_ref,
                  ws_ref, bs_ref, g1_ref, beta1_ref, m1_ref, v1_ref,
                  g2_ref, beta2_ref, m2_ref, v2_ref, wsh_ref, bsh_ref,
                  out_ref,
                  w1e, w2e, wsh5, rowp):
    i = pl.program_id(0)

    @pl.when(i == 0)
    def _prep():
        w2e[...] = jnp.zeros((224, _ED), jnp.float32)
        rowp[...] = jnp.zeros((8, 224), jnp.float32)
        rr = jax.lax.broadcasted_iota(jnp.int32, (_D, _D), 0)
        cc = jax.lax.broadcasted_iota(jnp.int32, (_D, _D), 1)
        ey = jnp.where(rr == cc, 1.0, 0.0).astype(jnp.float32)
        w1e[:, pl.ds(160, _D)] = ey
        w1e[:, pl.ds(192, _D)] = -ey
        g1t = jnp.transpose(g1_ref[...])
        v1t = jnp.transpose(v1_ref[...])
        for e in range(_E):
            sl = pl.ds(e * _D, _D)
            w1e[:, sl] = w1_ref[e]
            a1c = g1t[:, e:e + 1] * jax.lax.rsqrt(v1t[:, e:e + 1] + _EPS)
            a1r = g1_ref[e:e + 1, :] * jax.lax.rsqrt(v1_ref[e:e + 1, :] + _EPS)
            c1r = beta1_ref[e:e + 1, :] - m1_ref[e:e + 1, :] * a1r
            w2orig = w2_ref[e]
            w2e[sl, sl] = a1c * w2orig
            w2e[pl.ds(160, _D), sl] = ws_ref[e]
            w2e[pl.ds(192, _D), sl] = -ws_ref[e]
            b2eff = b2_ref[e:e + 1, :] + bs_ref[e:e + 1, :] + \
                jnp.dot(c1r, w2orig, preferred_element_type=jnp.float32)
            a2 = g2_ref[e:e + 1, :] * jax.lax.rsqrt(v2_ref[e:e + 1, :] + _EPS)
            c2 = beta2_ref[e:e + 1, :] - m2_ref[e:e + 1, :] * a2
            rowp[0:1, sl] = b1_ref[e:e + 1, :]
            rowp[1:2, sl] = b2eff
            rowp[2:3, sl] = a2
            rowp[3:4, sl] = c2
            wsh5[sl, :] = wsh_ref[...] * _BRANCH_WEIGHT

    x = x_ref[...]
    h1 = jnp.maximum(
        jnp.dot(x, w1e[...], preferred_element_type=jnp.float32)
        + rowp[0:1, :], 0.0)
    z2 = jnp.dot(h1, w2e[...], preferred_element_type=jnp.float32) \
        + rowp[1:2, 0:_ED]
    h2 = rowp[2:3, 0:_ED] * jnp.maximum(z2, 0.0) + rowp[3:4, 0:_ED]
    spt = jnp.transpose(sp_ref[...])
    group = jax.lax.broadcasted_iota(jnp.int32, (128, _ED), 1) // _D
    nchunks = h2.shape[0] // 128
    h2m = jnp.concatenate(
        [jnp.where(spt[:, r:r + 1] == group,
                   h2[r * 128:(r + 1) * 128, :], 0.0)
         for r in range(nchunks)], axis=0)
    out = jnp.dot(h2m, wsh5[...], preferred_element_type=jnp.float32) \
        + bsh_ref[...][None, :] * _BRANCH_WEIGHT
    out_ref[...] = jnp.maximum(out, 0.0)


def kernel(network_feat, species_ids, W1, b1, W2, b2, Ws, bs,
           g1, beta1, m1, v1, g2, beta2, m2, v2, Wsh, bsh):
    B, D = network_feat.shape
    assert D == _D
    f32 = jnp.float32

    sp_packed = species_ids.astype(jnp.int32).reshape(B // 128, 128)

    Bb = 2048
    grid = (B // Bb,)
    ew = lambda: pl.BlockSpec((_E, D, D), lambda i: (0, 0, 0))
    ev = lambda: pl.BlockSpec((_E, D), lambda i: (0, 0))
    out = pl.pallas_call(
        _branch_block,
        grid=grid,
        in_specs=[
            pl.BlockSpec((Bb // 128, 128), lambda i: (i, 0)),
            pl.BlockSpec((Bb, D), lambda i: (i, 0)),
            ew(), ev(),
            ew(), ev(),
            ew(), ev(),
            ev(), ev(), ev(), ev(),
            ev(), ev(), ev(), ev(),
            pl.BlockSpec((D, D), lambda i: (0, 0)),
            pl.BlockSpec((D,), lambda i: (0,)),
        ],
        out_specs=pl.BlockSpec((Bb, D), lambda i: (i, 0)),
        out_shape=jax.ShapeDtypeStruct((B, D), f32),
        scratch_shapes=[
            pltpu.VMEM((D, 224), f32),
            pltpu.VMEM((224, _ED), f32),
            pltpu.VMEM((_ED, D), f32),
            pltpu.VMEM((8, 224), f32),
        ],
        compiler_params=pltpu.CompilerParams(
            dimension_semantics=("arbitrary",)),
    )(sp_packed, network_feat.astype(f32), W1, b1, W2, b2, Ws, bs,
      g1, beta1, m1, v1, g2, beta2, m2, v2, Wsh, bsh)
    return out


# speedup vs baseline: 1.0738x; 1.0738x over previous
import jax
import jax.numpy as jnp
from jax.experimental import pallas as pl
from jax.experimental.pallas import tpu as pltpu

_E = 5
_D = 32
_ED = _E * _D
_EPS = 1e-5
_BRANCH_WEIGHT = 0.92


def _branch_block(sp_ref, x_ref, w1_ref, b1_ref, w2_ref, b2_ref,
                  ws_ref, bs_ref, g1_ref, beta1_ref, m1_ref, v1_ref,
                  g2_ref, beta2_ref, m2_ref, v2_ref, wsh_ref, bsh_ref,
                  out_ref,
                  w1e, w2e, wsh5, rowp):
    i = pl.program_id(0)

    @pl.when(i == 0)
    def _prep():
        w2e[...] = jnp.zeros((224, _ED), jnp.float32)
        rowp[...] = jnp.zeros((8, 224), jnp.float32)
        rr = jax.lax.broadcasted_iota(jnp.int32, (_D, _D), 0)
        cc = jax.lax.broadcasted_iota(jnp.int32, (_D, _D), 1)
        ey = jnp.where(rr == cc, 1.0, 0.0).astype(jnp.float32)
        w1e[:, pl.ds(160, _D)] = ey
        w1e[:, pl.ds(192, _D)] = -ey
        g1t = jnp.transpose(g1_ref[...])
        v1t = jnp.transpose(v1_ref[...])
        for e in range(_E):
            sl = pl.ds(e * _D, _D)
            w1e[:, sl] = w1_ref[e]
            a1c = g1t[:, e:e + 1] * jax.lax.rsqrt(v1t[:, e:e + 1] + _EPS)
            a1r = g1_ref[e:e + 1, :] * jax.lax.rsqrt(v1_ref[e:e + 1, :] + _EPS)
            c1r = beta1_ref[e:e + 1, :] - m1_ref[e:e + 1, :] * a1r
            w2orig = w2_ref[e]
            w2e[sl, sl] = a1c * w2orig
            w2e[pl.ds(160, _D), sl] = ws_ref[e]
            w2e[pl.ds(192, _D), sl] = -ws_ref[e]
            b2eff = b2_ref[e:e + 1, :] + bs_ref[e:e + 1, :] + \
                jnp.dot(c1r, w2orig, preferred_element_type=jnp.float32)
            a2 = g2_ref[e:e + 1, :] * jax.lax.rsqrt(v2_ref[e:e + 1, :] + _EPS)
            c2 = beta2_ref[e:e + 1, :] - m2_ref[e:e + 1, :] * a2
            rowp[0:1, sl] = b1_ref[e:e + 1, :]
            rowp[1:2, sl] = b2eff
            rowp[2:3, sl] = a2
            rowp[3:4, sl] = c2
            wsh5[sl, :] = wsh_ref[...] * _BRANCH_WEIGHT

    x = x_ref[...]
    h1 = jnp.maximum(
        jnp.dot(x, w1e[...], preferred_element_type=jnp.float32)
        + rowp[0:1, :], 0.0)
    z2 = jnp.dot(h1, w2e[...], preferred_element_type=jnp.float32) \
        + rowp[1:2, 0:_ED]
    h2 = rowp[2:3, 0:_ED] * jnp.maximum(z2, 0.0) + rowp[3:4, 0:_ED]
    spt = jnp.transpose(sp_ref[...])
    group = jax.lax.broadcasted_iota(jnp.int32, (128, _ED), 1) // _D
    nchunks = h2.shape[0] // 128
    h2m = jnp.concatenate(
        [jnp.where(spt[:, r:r + 1] == group,
                   h2[r * 128:(r + 1) * 128, :], 0.0)
         for r in range(nchunks)], axis=0)
    out = jnp.dot(h2m, wsh5[...], preferred_element_type=jnp.float32) \
        + bsh_ref[...][None, :] * _BRANCH_WEIGHT
    out_ref[...] = jnp.maximum(out, 0.0)


def kernel(network_feat, species_ids, W1, b1, W2, b2, Ws, bs,
           g1, beta1, m1, v1, g2, beta2, m2, v2, Wsh, bsh):
    B, D = network_feat.shape
    assert D == _D
    f32 = jnp.float32

    sp_packed = species_ids.astype(jnp.int32).reshape(B // 128, 128)

    Bb = 4096
    grid = (B // Bb,)
    ew = lambda: pl.BlockSpec((_E, D, D), lambda i: (0, 0, 0))
    ev = lambda: pl.BlockSpec((_E, D), lambda i: (0, 0))
    out = pl.pallas_call(
        _branch_block,
        grid=grid,
        in_specs=[
            pl.BlockSpec((Bb // 128, 128), lambda i: (i, 0)),
            pl.BlockSpec((Bb, D), lambda i: (i, 0)),
            ew(), ev(),
            ew(), ev(),
            ew(), ev(),
            ev(), ev(), ev(), ev(),
            ev(), ev(), ev(), ev(),
            pl.BlockSpec((D, D), lambda i: (0, 0)),
            pl.BlockSpec((D,), lambda i: (0,)),
        ],
        out_specs=pl.BlockSpec((Bb, D), lambda i: (i, 0)),
        out_shape=jax.ShapeDtypeStruct((B, D), f32),
        scratch_shapes=[
            pltpu.VMEM((D, 224), f32),
            pltpu.VMEM((224, _ED), f32),
            pltpu.VMEM((_ED, D), f32),
            pltpu.VMEM((8, 224), f32),
        ],
        compiler_params=pltpu.CompilerParams(
            dimension_semantics=("arbitrary",)),
    )(sp_packed, network_feat.astype(f32), W1, b1, W2, b2, Ws, bs,
      g1, beta1, m1, v1, g2, beta2, m2, v2, Wsh, bsh)
    return out
